# baseline (speedup 1.0000x reference)
import functools

import jax
import jax.numpy as jnp
from jax import lax
from jax.experimental import pallas as pl
from jax.experimental.pallas import tpu as pltpu
from jax.experimental.pallas import tpu_sc as plsc

N = 10000
NPAD = 10240
E = 320000
H = 128
NBASIS = 8
RMAX = 5.0

NC = 2
NS = 16
NWORK = NC * NS
CB = 64
NCHUNK = 157
EW = NCHUNK * CB
EPAD = NWORK * EW
DUMMY = 10015
NACC = 10112
ROWS_T = NACC // NS
DROWS_T = NPAD // NS

EBLK = 2048
NBLK = 1024
F32 = jnp.float32


def _mesh():
    return plsc.VectorSubcoreMesh(core_axis_name="c", subcore_axis_name="s",
                                  num_cores=NC, num_subcores=NS)


def _prologue_body(pos16, eidx, zeros16, posg, degp,
                   idx, rows_s, rows_d, ones_v, degacc,
                   semi, semg, semw, semo):
    c = lax.axis_index("c")
    s = lax.axis_index("s")
    wid = c * NS + s
    base0 = wid * EW
    pltpu.sync_copy(zeros16.at[pl.ds(s * DROWS_T, DROWS_T)],
                    degacc.at[pl.ds(s * DROWS_T, DROWS_T)])
    def fill_ones(r, _):
        ones_v[r] = jnp.full((16,), 1.0, F32)
        return 0
    lax.fori_loop(0, CB, fill_ones, 0)
    plsc.subcore_barrier()

    def step(jj, _):
        for p in range(NBUF):
            j = jj * NBUF + p

            b3 = (p - 3) % NBUF
            @pl.when(jnp.logical_and(j - 3 >= 0, j - 3 < NCHUNK))
            def _(b=b3):
                pltpu.make_async_copy(rows_s.at[b], posg.at[0, pl.ds(0, CB)],
                                      semw.at[b]).wait()
                pltpu.make_async_copy(rows_d.at[b], posg.at[1, pl.ds(0, CB)],
                                      semw.at[b]).wait()
                pltpu.make_async_copy(ones_v, degacc.at[idx.at[b, 1]],
                                      semo.at[b]).wait()

            @pl.when(j < NCHUNK)
            def _(p=p, j=j):
                pltpu.async_copy(eidx.at[wid, j], idx.at[p], semi.at[p])

            b1 = (p - 1) % NBUF
            @pl.when(jnp.logical_and(j - 1 >= 0, j - 1 < NCHUNK))
            def _(b=b1):
                pltpu.make_async_copy(eidx.at[0, 0], idx.at[b],
                                      semi.at[b]).wait()
                pltpu.async_copy(pos16.at[idx.at[b, 0]], rows_s.at[b],
                                 semg.at[b])
                pltpu.async_copy(pos16.at[idx.at[b, 1]], rows_d.at[b],
                                 semg.at[b])

            b2 = (p - 2) % NBUF
            @pl.when(jnp.logical_and(j - 2 >= 0, j - 2 < NCHUNK))
            def _(b=b2, j=j):
                base = base0 + (j - 2) * CB
                pltpu.make_async_copy(pos16.at[idx.at[b, 0]], rows_s.at[b],
                                      semg.at[b]).wait()
                pltpu.make_async_copy(pos16.at[idx.at[b, 1]], rows_d.at[b],
                                      semg.at[b]).wait()
                pltpu.async_copy(rows_s.at[b], posg.at[0, pl.ds(base, CB)],
                                 semw.at[b])
                pltpu.async_copy(rows_d.at[b], posg.at[1, pl.ds(base, CB)],
                                 semw.at[b])
                pltpu.async_copy(ones_v, degacc.at[idx.at[b, 1]],
                                 semo.at[b], add=True)
        return 0

    lax.fori_loop(0, (NCHUNK + 3) // NBUF, step, 0)
    plsc.subcore_barrier()
    pltpu.sync_copy(degacc.at[pl.ds(s * DROWS_T, DROWS_T)],
                    degp.at[c, pl.ds(s * DROWS_T, DROWS_T)])


def _sc_prologue(pos16, eidx, zeros16, interpret=False):
    fn = pl.kernel(
        _prologue_body,
        out_type=(jax.ShapeDtypeStruct((2, EPAD, 16), F32),
                  jax.ShapeDtypeStruct((2, NPAD, 16), F32)),
        mesh=_mesh(),
        scratch_types=[
            pltpu.VMEM((NBUF, 2, CB), jnp.int32),
            pltpu.VMEM((NBUF, CB, 16), F32),
            pltpu.VMEM((NBUF, CB, 16), F32),
            pltpu.VMEM((CB, 16), F32),
            pltpu.VMEM_SHARED((NPAD, 16), F32),
            pltpu.SemaphoreType.DMA((NBUF,)),
            pltpu.SemaphoreType.DMA((NBUF,)),
            pltpu.SemaphoreType.DMA((NBUF,)),
            pltpu.SemaphoreType.DMA((NBUF,)),
        ],
        compiler_params=pltpu.CompilerParams(use_tc_tiling_on_sc=False),
        interpret=interpret,
    )
    return fn(pos16, eidx, zeros16)


NBUF = 4
NSTEP = NCHUNK + 3


def _edge_body(hA, hB, efc, eidx, zerosN, sp,
               idx, rows, efp, acc, semi, sema, semb, seme, sems):
    c = lax.axis_index("c")
    s = lax.axis_index("s")
    wid = c * NS + s
    base0 = wid * EW

    pltpu.sync_copy(zerosN.at[pl.ds(s * ROWS_T, ROWS_T)],
                    acc.at[pl.ds(s * ROWS_T, ROWS_T)])
    plsc.subcore_barrier()

    def step(jj, _):
        for p in range(NBUF):
            j = jj * NBUF + p

            @pl.when(j - NBUF >= 0)
            def _(p=p):
                pltpu.make_async_copy(rows.at[p], acc.at[idx.at[p, 1]],
                                      sems.at[p]).wait()

            @pl.when(j < NCHUNK)
            def _(p=p, j=j):
                base = pl.multiple_of(base0 + j * CB, 8)
                pltpu.async_copy(eidx.at[wid, j], idx.at[p], semi.at[p])
                pltpu.async_copy(efc.at[pl.ds(base, CB)], efp.at[p],
                                 seme.at[p])

            b1 = (p - 1) % NBUF
            @pl.when(jnp.logical_and(j - 1 >= 0, j - 1 < NCHUNK))
            def _(b=b1):
                pltpu.make_async_copy(eidx.at[0, 0], idx.at[b],
                                      semi.at[b]).wait()
                pltpu.async_copy(hA.at[idx.at[b, 1]], rows.at[b], sema.at[b])

            b2 = (p - 2) % NBUF
            @pl.when(jnp.logical_and(j - 2 >= 0, j - 2 < NCHUNK))
            def _(b=b2):
                pltpu.make_async_copy(hA.at[idx.at[b, 1]], rows.at[b],
                                      sema.at[b]).wait()
                pltpu.async_copy(hB.at[idx.at[b, 0]], rows.at[b], semb.at[b],
                                 add=True)

            b3 = (p - 3) % NBUF
            @pl.when(jnp.logical_and(j - 3 >= 0, j - 3 < NCHUNK))
            def _(b=b3):
                pltpu.make_async_copy(hB.at[idx.at[b, 0]], rows.at[b],
                                      semb.at[b]).wait()
                pltpu.make_async_copy(efc.at[pl.ds(0, CB)], efp.at[b],
                                      seme.at[b]).wait()

                def fuse_row(e, _):
                    for k in range(H // 32):
                        sl = pl.ds(k * 16, 16)
                        sh = pl.ds(64 + k * 16, 16)
                        w = efp[b, e, pl.ds(k * 16, 16)]
                        flo = plsc.bitcast(w << 16, F32)
                        fhi = plsc.bitcast(w & jnp.int32(-65536), F32)
                        rows[b, e, sl] = jnp.maximum(rows[b, e, sl] + flo, 0.0)
                        rows[b, e, sh] = jnp.maximum(rows[b, e, sh] + fhi, 0.0)
                    return 0
                lax.fori_loop(0, CB, fuse_row, 0)
                pltpu.async_copy(rows.at[b], acc.at[idx.at[b, 1]], sems.at[b],
                                 add=True)
        return 0

    lax.fori_loop(0, NSTEP // NBUF, step, 0)
    pltpu.make_async_copy(rows.at[(NCHUNK - 1) % NBUF],
                          acc.at[idx.at[(NCHUNK - 1) % NBUF, 1]],
                          sems.at[(NCHUNK - 1) % NBUF]).wait()
    plsc.subcore_barrier()

    pltpu.sync_copy(acc.at[pl.ds(s * ROWS_T, ROWS_T)],
                    sp.at[c, pl.ds(s * ROWS_T, ROWS_T)])


def _sc_edge(hA, hB, efc, eidx, zerosN, interpret=False):
    fn = pl.kernel(
        _edge_body,
        out_type=jax.ShapeDtypeStruct((2, NACC, H), F32),
        mesh=_mesh(),
        scratch_types=[
            pltpu.VMEM((NBUF, 2, CB), jnp.int32),
            pltpu.VMEM((NBUF, CB, H), F32),
            pltpu.VMEM((NBUF, CB, H // 2), jnp.int32),
            pltpu.VMEM_SHARED((NACC, H), F32),
            pltpu.SemaphoreType.DMA((NBUF,)),
            pltpu.SemaphoreType.DMA((NBUF,)),
            pltpu.SemaphoreType.DMA((NBUF,)),
            pltpu.SemaphoreType.DMA((NBUF,)),
            pltpu.SemaphoreType.DMA((NBUF,)),
        ],
        compiler_params=pltpu.CompilerParams(use_tc_tiling_on_sc=False,
                                             needs_layout_passes=False),
        interpret=interpret,
    )
    return fn(hA, hB, efc, eidx, zerosN)


def _embed_body(x_ref, w_ref, b_ref, ab_ref, o_ref, oa_ref, ob_ref):
    h = (jnp.dot(x_ref[...], w_ref[...], preferred_element_type=F32)
         + b_ref[...])
    o_ref[...] = h
    oa_ref[...] = jnp.dot(h, ab_ref[0], preferred_element_type=F32)
    ob_ref[...] = jnp.dot(h, ab_ref[1], preferred_element_type=F32)


def _tc_embed(x16, Wn16, bn, AB0, interpret=False):
    return pl.pallas_call(
        _embed_body,
        grid=(NPAD // NBLK,),
        in_specs=[pl.BlockSpec((NBLK, 16), lambda i: (i, 0)),
                  pl.BlockSpec((16, H), lambda i: (0, 0)),
                  pl.BlockSpec((1, H), lambda i: (0, 0)),
                  pl.BlockSpec((2, H, H), lambda i: (0, 0, 0))],
        out_specs=[pl.BlockSpec((NBLK, H), lambda i: (i, 0)),
                   pl.BlockSpec((NBLK, H), lambda i: (i, 0)),
                   pl.BlockSpec((NBLK, H), lambda i: (i, 0))],
        out_shape=[jax.ShapeDtypeStruct((NPAD, H), F32),
                   jax.ShapeDtypeStruct((NPAD, H), F32),
                   jax.ShapeDtypeStruct((NPAD, H), F32)],
        interpret=interpret,
    )(x16, Wn16, bn, AB0)


def _basis_body(pg_ref, ea_ref, o_ref):
    ps = pg_ref[0]
    pd = pg_ref[1]
    d = pd - ps
    lane = lax.broadcasted_iota(jnp.int32, (EBLK, 16), 1)
    d = jnp.where(lane < 3, d, 0.0)
    lensq = jnp.sum(d * d, axis=1, keepdims=True)
    ln = jnp.sqrt(lensq + 1e-12)
    wdt = RMAX / (NBASIS - 1)
    centers = (lane.astype(F32) - 3.0) * wdt
    bas = jnp.exp(-0.5 * ((ln - centers) / wdt) ** 2)
    o_ref[...] = jnp.where(lane < 3, ea_ref[...],
                           jnp.where(lane < 11, bas, 0.0))


def _tc_basis(posg, ea16, interpret=False):
    return pl.pallas_call(
        _basis_body,
        grid=(EPAD // EBLK,),
        in_specs=[pl.BlockSpec((2, EBLK, 16), lambda i: (0, i, 0)),
                  pl.BlockSpec((EBLK, 16), lambda i: (i, 0))],
        out_specs=pl.BlockSpec((EBLK, 16), lambda i: (i, 0)),
        out_shape=jax.ShapeDtypeStruct((EPAD, 16), F32),
        interpret=interpret,
    )(posg, ea16)


def _efc_body(r_ref, d_ref, dv_ref, o_ref):
    val = (jnp.dot(r_ref[...], d_ref[...], preferred_element_type=F32)
           + dv_ref[...])
    rb = lax.bitcast_convert_type(val, jnp.int32) + jnp.int32(0x8000)
    lo = (rb[:, 0:H // 2] >> 16) & jnp.int32(0xFFFF)
    hi = rb[:, H // 2:] & jnp.int32(-65536)
    o_ref[...] = lo | hi


def _tc_efc(raw, D16, dvec, interpret=False):
    return pl.pallas_call(
        _efc_body,
        grid=(EPAD // EBLK,),
        in_specs=[pl.BlockSpec((EBLK, 16), lambda i: (i, 0)),
                  pl.BlockSpec((16, H), lambda i: (0, 0)),
                  pl.BlockSpec((1, H), lambda i: (0, 0))],
        out_specs=pl.BlockSpec((EBLK, H // 2), lambda i: (i, 0)),
        out_shape=jax.ShapeDtypeStruct((EPAD, H // 2), jnp.int32),
        interpret=interpret,
    )(raw, D16, dvec)


def _update_body(h_ref, sp_ref, degp_ref, w_ref, v_ref, o_ref, oa_ref, ob_ref):
    h = h_ref[...]
    S = sp_ref[0] + sp_ref[1]
    deg = degp_ref[0][:, 0:1] + degp_ref[1][:, 0:1]
    W_m2 = w_ref[0]
    U1h = w_ref[1]
    U1a = w_ref[2]
    W_u2 = w_ref[3]
    b_m2 = v_ref[0:1]
    b_u1 = v_ref[1:2]
    b_u2 = v_ref[2:3]
    gam = v_ref[3:4]
    bet = v_ref[4:5]
    agg = jnp.dot(S, W_m2, preferred_element_type=F32) + deg * b_m2
    u = jnp.maximum(jnp.dot(h, U1h, preferred_element_type=F32)
                    + jnp.dot(agg, U1a, preferred_element_type=F32)
                    + b_u1, 0.0)
    out = jnp.dot(u, W_u2, preferred_element_type=F32) + b_u2 + h
    mu = jnp.mean(out, axis=1, keepdims=True)
    var = jnp.mean((out - mu) ** 2, axis=1, keepdims=True)
    hn = (out - mu) * lax.rsqrt(var + 1e-5) * gam + bet
    o_ref[...] = hn
    oa_ref[...] = jnp.dot(hn, w_ref[4], preferred_element_type=F32)
    ob_ref[...] = jnp.dot(hn, w_ref[5], preferred_element_type=F32)


def _tc_update(h, Sp, degp, Wm, vecs, interpret=False):
    return pl.pallas_call(
        _update_body,
        grid=(NPAD // NBLK,),
        in_specs=[pl.BlockSpec((NBLK, H), lambda i: (i, 0)),
                  pl.BlockSpec((2, NBLK, H), lambda i: (0, i, 0)),
                  pl.BlockSpec((2, NBLK, 16), lambda i: (0, i, 0)),
                  pl.BlockSpec((6, H, H), lambda i: (0, 0, 0)),
                  pl.BlockSpec((8, H), lambda i: (0, 0))],
        out_specs=[pl.BlockSpec((NBLK, H), lambda i: (i, 0)),
                   pl.BlockSpec((NBLK, H), lambda i: (i, 0)),
                   pl.BlockSpec((NBLK, H), lambda i: (i, 0))],
        out_shape=[jax.ShapeDtypeStruct((NPAD, H), F32),
                   jax.ShapeDtypeStruct((NPAD, H), F32),
                   jax.ShapeDtypeStruct((NPAD, H), F32)],
        interpret=interpret,
    )(h, Sp, degp, Wm, vecs)


def _head_body(h_ref, b3_ref, w_ref, v_ref, o_ref, acc_ref):
    i = pl.program_id(0)

    @pl.when(i == 0)
    def _init():
        acc_ref[...] = jnp.zeros_like(acc_ref)

    h = h_ref[...]
    b = b3_ref[0]
    m1 = (b == 0).astype(F32)
    m2 = (b == 1).astype(F32)
    acc_ref[0:1] += jnp.dot(m1, h, preferred_element_type=F32)
    acc_ref[1:2] += jnp.dot(m2, h, preferred_element_type=F32)
    acc_ref[2:3] = acc_ref[2:3] + jnp.sum(m1)
    acc_ref[3:4] = acc_ref[3:4] + jnp.sum(m2)

    @pl.when(i == NPAD // NBLK - 1)
    def _fin():
        c1 = jnp.maximum(acc_ref[2:3], 1.0)
        c2 = jnp.maximum(acc_ref[3:4], 1.0)
        s1 = acc_ref[0:1] / c1
        s2 = acc_ref[1:2] / c2
        hh = jnp.maximum(jnp.dot(s1, w_ref[0], preferred_element_type=F32)
                         + jnp.dot(s2, w_ref[1], preferred_element_type=F32)
                         + v_ref[0:1], 0.0)
        hh = jnp.maximum(jnp.dot(hh, w_ref[2], preferred_element_type=F32)
                         + v_ref[1:2], 0.0)
        z = jnp.dot(hh, w_ref[3], preferred_element_type=F32) + v_ref[2:3]
        o_ref[...] = jnp.maximum(z, 0.0) + jnp.log(1.0 + jnp.exp(-jnp.abs(z)))


def _tc_head(h, batch3, Whead, vhead, interpret=False):
    return pl.pallas_call(
        _head_body,
        grid=(NPAD // NBLK,),
        in_specs=[pl.BlockSpec((NBLK, H), lambda i: (i, 0)),
                  pl.BlockSpec((1, 1, NBLK), lambda i: (i, 0, 0)),
                  pl.BlockSpec((4, H, H), lambda i: (0, 0, 0)),
                  pl.BlockSpec((8, H), lambda i: (0, 0))],
        out_specs=pl.BlockSpec((1, H), lambda i: (0, 0)),
        out_shape=jax.ShapeDtypeStruct((1, H), F32),
        scratch_shapes=[pltpu.VMEM((8, H), F32)],
        interpret=interpret,
    )(h, batch3, Whead, vhead)


def _forward(x, edge_index, edge_attr, pos, batch, params, interpret=False):
    srcI = edge_index[0].astype(jnp.int32)
    dstI = edge_index[1].astype(jnp.int32)
    x16 = jnp.zeros((NPAD, 16), F32).at[:N, :6].set(x)
    pos16 = jnp.zeros((NPAD, 16), F32).at[:N, :3].set(pos)
    eaw = jnp.zeros((NWORK, EW, 3), F32).at[:, :E // NWORK].set(
        edge_attr.reshape(NWORK, E // NWORK, 3))
    ea16 = jnp.zeros((EPAD, 16), F32).at[:, :3].set(eaw.reshape(EPAD, 3))
    batchp = jnp.full((NPAD,), 2, jnp.int32).at[:N].set(batch.astype(jnp.int32))
    batch3 = batchp.reshape(NPAD // NBLK, 1, NBLK)
    zeros16 = jnp.zeros((NPAD, 16), F32)
    zerosN = jnp.zeros((NACC, H), F32)

    Wn16 = jnp.zeros((16, H), F32).at[:6].set(params['W_node'])
    bn = params['b_node'][None]
    We = params['W_edge']
    be = params['b_edge']
    z128 = jnp.zeros((H,), F32)
    zHH = jnp.zeros((H, H), F32)
    mp = params['mp']
    nl = len(mp)
    As = [lp['W_m1'][0:H] for lp in mp]
    Bs = [lp['W_m1'][H:2 * H] for lp in mp]
    Ds, dvs, Wms, vls = [], [], [], []
    for l, lp in enumerate(mp):
        C = lp['W_m1'][2 * H:]
        Ds.append(jnp.zeros((16, H), F32).at[:11].set(We @ C))
        dvs.append((be @ C + lp['b_m1'])[None])
        An = As[l + 1] if l + 1 < nl else zHH
        Bn = Bs[l + 1] if l + 1 < nl else zHH
        Wms.append(jnp.stack([lp['W_m2'], lp['W_u1'][0:H],
                              lp['W_u1'][H:], lp['W_u2'], An, Bn]))
        vls.append(jnp.stack([lp['b_m2'], lp['b_u1'], lp['b_u2'],
                              lp['gamma'], lp['beta'], z128, z128, z128]))
    D16s = jnp.stack(Ds)
    dvecs = jnp.stack(dvs)
    AB0 = jnp.stack([As[0], Bs[0]])
    srcw = jnp.full((NWORK, EW), DUMMY, jnp.int32).at[:, :E // NWORK].set(
        srcI.reshape(NWORK, E // NWORK))
    dstw = jnp.full((NWORK, EW), DUMMY, jnp.int32).at[:, :E // NWORK].set(
        dstI.reshape(NWORK, E // NWORK))
    eidx = jnp.stack([srcw.reshape(NWORK, NCHUNK, CB),
                      dstw.reshape(NWORK, NCHUNK, CB)], axis=2)
    Wh2 = jnp.zeros((H, H), F32).at[:, :H // 2].set(params['W_h2'])
    Wh3 = jnp.zeros((H, H), F32).at[:H // 2, 0:1].set(params['W_h3'])
    Whead = jnp.stack([params['W_h1'][0:H], params['W_h1'][H:], Wh2, Wh3])
    bh2p = jnp.zeros((H,), F32).at[:H // 2].set(params['b_h2'])
    vhead = jnp.stack([params['b_h1'], bh2p,
                       jnp.broadcast_to(params['b_h3'], (H,)),
                       z128, z128, z128, z128, z128])

    h, hA, hB = _tc_embed(x16, Wn16, bn, AB0, interpret)
    posg, degp = _sc_prologue(pos16, eidx, zeros16, interpret)
    raw = _tc_basis(posg, ea16, interpret)
    efc = _tc_efc(raw, D16s[0], dvecs[0], interpret)
    for l in range(nl):
        Sp = _sc_edge(hA, hB, efc, eidx, zerosN, interpret)
        if l + 1 < nl:
            efc = _tc_efc(raw, D16s[l + 1], dvecs[l + 1], interpret)
        Spp = jnp.pad(Sp, ((0, 0), (0, NPAD - NACC), (0, 0)))
        h, hA, hB = _tc_update(h, Spp, degp, Wms[l], vls[l], interpret)
    out = _tc_head(h, batch3, Whead, vhead, interpret)
    return out[0, 0:1]


def kernel(x, edge_index, edge_attr, pos, batch, params):
    return _forward(x, edge_index, edge_attr, pos, batch, params)


# speedup vs baseline: 1.3810x; 1.3810x over previous
import functools

import jax
import jax.numpy as jnp
from jax import lax
from jax.experimental import pallas as pl
from jax.experimental.pallas import tpu as pltpu
from jax.experimental.pallas import tpu_sc as plsc

N = 10000
NPAD = 10240
E = 320000
H = 128
NBASIS = 8
RMAX = 5.0

NC = 2
NS = 16
NWORK = NC * NS
EW = E // NWORK
CB = 80
NCHUNK = EW // CB
ROWS_T = NPAD // NS

EBLK = 4000
NBLK = 1024
F32 = jnp.float32


def _mesh():
    return plsc.VectorSubcoreMesh(core_axis_name="c", subcore_axis_name="s",
                                  num_cores=NC, num_subcores=NS)


def _prologue_body(pos16, eidx, zeros16, posg, degp,
                   idx, rows_s, rows_d, ones_v, degacc,
                   semi, semg, semw, semo):
    c = lax.axis_index("c")
    s = lax.axis_index("s")
    wid = c * NS + s
    base0 = wid * EW
    pltpu.sync_copy(zeros16.at[pl.ds(s * ROWS_T, ROWS_T)],
                    degacc.at[pl.ds(s * ROWS_T, ROWS_T)])
    def fill_ones(r, _):
        ones_v[r] = jnp.full((16,), 1.0, F32)
        return 0
    lax.fori_loop(0, CB, fill_ones, 0)
    plsc.subcore_barrier()

    def step(jj, _):
        for p in range(NBUF):
            j = jj * NBUF + p

            b3 = (p - 3) % NBUF
            @pl.when(jnp.logical_and(j - 3 >= 0, j - 3 < NCHUNK))
            def _(b=b3):
                pltpu.make_async_copy(rows_s.at[b], posg.at[0, pl.ds(0, CB)],
                                      semw.at[b]).wait()
                pltpu.make_async_copy(rows_d.at[b], posg.at[1, pl.ds(0, CB)],
                                      semw.at[b]).wait()
                pltpu.make_async_copy(ones_v, degacc.at[idx.at[b, 1]],
                                      semo.at[b]).wait()

            @pl.when(j < NCHUNK)
            def _(p=p, j=j):
                pltpu.async_copy(eidx.at[wid, j], idx.at[p], semi.at[p])

            b1 = (p - 1) % NBUF
            @pl.when(jnp.logical_and(j - 1 >= 0, j - 1 < NCHUNK))
            def _(b=b1):
                pltpu.make_async_copy(eidx.at[0, 0], idx.at[b],
                                      semi.at[b]).wait()
                pltpu.async_copy(pos16.at[idx.at[b, 0]], rows_s.at[b],
                                 semg.at[b])
                pltpu.async_copy(pos16.at[idx.at[b, 1]], rows_d.at[b],
                                 semg.at[b])

            b2 = (p - 2) % NBUF
            @pl.when(jnp.logical_and(j - 2 >= 0, j - 2 < NCHUNK))
            def _(b=b2, j=j):
                base = base0 + (j - 2) * CB
                pltpu.make_async_copy(pos16.at[idx.at[b, 0]], rows_s.at[b],
                                      semg.at[b]).wait()
                pltpu.make_async_copy(pos16.at[idx.at[b, 1]], rows_d.at[b],
                                      semg.at[b]).wait()
                pltpu.async_copy(rows_s.at[b], posg.at[0, pl.ds(base, CB)],
                                 semw.at[b])
                pltpu.async_copy(rows_d.at[b], posg.at[1, pl.ds(base, CB)],
                                 semw.at[b])
                pltpu.async_copy(ones_v, degacc.at[idx.at[b, 1]],
                                 semo.at[b], add=True)
        return 0

    lax.fori_loop(0, (NCHUNK + 3) // NBUF, step, 0)
    plsc.subcore_barrier()
    pltpu.sync_copy(degacc.at[pl.ds(s * ROWS_T, ROWS_T)],
                    degp.at[c, pl.ds(s * ROWS_T, ROWS_T)])


def _sc_prologue(pos16, eidx, zeros16, interpret=False):
    fn = pl.kernel(
        _prologue_body,
        out_type=(jax.ShapeDtypeStruct((2, E, 16), F32),
                  jax.ShapeDtypeStruct((2, NPAD, 16), F32)),
        mesh=_mesh(),
        scratch_types=[
            pltpu.VMEM((NBUF, 2, CB), jnp.int32),
            pltpu.VMEM((NBUF, CB, 16), F32),
            pltpu.VMEM((NBUF, CB, 16), F32),
            pltpu.VMEM((CB, 16), F32),
            pltpu.VMEM_SHARED((NPAD, 16), F32),
            pltpu.SemaphoreType.DMA((NBUF,)),
            pltpu.SemaphoreType.DMA((NBUF,)),
            pltpu.SemaphoreType.DMA((NBUF,)),
            pltpu.SemaphoreType.DMA((NBUF,)),
        ],
        compiler_params=pltpu.CompilerParams(use_tc_tiling_on_sc=False),
        interpret=interpret,
    )
    return fn(pos16, eidx, zeros16)


NBUF = 4
NSTEP = NCHUNK + 3


def _edge_body(loff, hA, hB, efc, eidx, zeros128, sp,
               idx, rows, acc, semi, sema, semb, seme, sems):
    c = lax.axis_index("c")
    s = lax.axis_index("s")
    wid = c * NS + s
    base0 = loff + wid * EW
    pltpu.sync_copy(zeros128.at[pl.ds(s * ROWS_T, ROWS_T)],
                    acc.at[pl.ds(s * ROWS_T, ROWS_T)])
    plsc.subcore_barrier()

    def step(jj, _):
        for p in range(NBUF):
            j = jj * NBUF + p

            @pl.when(j - NBUF >= 0)
            def _(p=p):
                pltpu.make_async_copy(rows.at[p], acc.at[idx.at[p, 1]],
                                      sems.at[p]).wait()

            @pl.when(j < NCHUNK)
            def _(p=p, j=j):
                base = base0 + j * CB
                pltpu.async_copy(eidx.at[wid, j], idx.at[p], semi.at[p])
                pltpu.async_copy(efc.at[pl.ds(base, CB)], rows.at[p],
                                 seme.at[p])

            b1 = (p - 1) % NBUF
            @pl.when(jnp.logical_and(j - 1 >= 0, j - 1 < NCHUNK))
            def _(b=b1):
                pltpu.make_async_copy(eidx.at[0, 0], idx.at[b],
                                      semi.at[b]).wait()
                pltpu.make_async_copy(efc.at[pl.ds(0, CB)], rows.at[b],
                                      seme.at[b]).wait()
                pltpu.async_copy(hA.at[idx.at[b, 1]], rows.at[b], sema.at[b],
                                 add=True)

            b2 = (p - 2) % NBUF
            @pl.when(jnp.logical_and(j - 2 >= 0, j - 2 < NCHUNK))
            def _(b=b2):
                pltpu.make_async_copy(hA.at[idx.at[b, 1]], rows.at[b],
                                      sema.at[b]).wait()
                pltpu.async_copy(hB.at[idx.at[b, 0]], rows.at[b], semb.at[b],
                                 add=True)

            b3 = (p - 3) % NBUF
            @pl.when(jnp.logical_and(j - 3 >= 0, j - 3 < NCHUNK))
            def _(b=b3):
                pltpu.make_async_copy(hB.at[idx.at[b, 0]], rows.at[b],
                                      semb.at[b]).wait()

                def relu_row(e, _):
                    for k in range(H // 16):
                        sl = pl.ds(k * 16, 16)
                        rows[b, e, sl] = jnp.maximum(rows[b, e, sl], 0.0)
                    return 0
                lax.fori_loop(0, CB, relu_row, 0)
                pltpu.async_copy(rows.at[b], acc.at[idx.at[b, 1]], sems.at[b],
                                 add=True)
        return 0

    lax.fori_loop(0, NSTEP // NBUF, step, 0)
    pltpu.make_async_copy(rows.at[(NCHUNK - 1) % NBUF],
                          acc.at[idx.at[(NCHUNK - 1) % NBUF, 1]],
                          sems.at[(NCHUNK - 1) % NBUF]).wait()
    plsc.subcore_barrier()
    pltpu.sync_copy(acc.at[pl.ds(s * ROWS_T, ROWS_T)],
                    sp.at[c, pl.ds(s * ROWS_T, ROWS_T)])


def _sc_edge(hA, hB, efc, eidx, zeros128, loff, interpret=False):
    fn = pl.kernel(
        functools.partial(_edge_body, loff),
        out_type=jax.ShapeDtypeStruct((2, NPAD, H), F32),
        mesh=_mesh(),
        scratch_types=[
            pltpu.VMEM((NBUF, 2, CB), jnp.int32),
            pltpu.VMEM((NBUF, CB, H), F32),
            pltpu.VMEM_SHARED((NPAD, H), F32),
            pltpu.SemaphoreType.DMA((NBUF,)),
            pltpu.SemaphoreType.DMA((NBUF,)),
            pltpu.SemaphoreType.DMA((NBUF,)),
            pltpu.SemaphoreType.DMA((NBUF,)),
            pltpu.SemaphoreType.DMA((NBUF,)),
        ],
        interpret=interpret,
    )
    return fn(hA, hB, efc, eidx, zeros128)


def _embed_body(x_ref, w_ref, b_ref, ab_ref, o_ref, oa_ref, ob_ref):
    h = (jnp.dot(x_ref[...], w_ref[...], preferred_element_type=F32)
         + b_ref[...])
    o_ref[...] = h
    oa_ref[...] = jnp.dot(h, ab_ref[0], preferred_element_type=F32)
    ob_ref[...] = jnp.dot(h, ab_ref[1], preferred_element_type=F32)


def _tc_embed(x16, Wn16, bn, AB0, interpret=False):
    return pl.pallas_call(
        _embed_body,
        grid=(NPAD // NBLK,),
        in_specs=[pl.BlockSpec((NBLK, 16), lambda i: (i, 0)),
                  pl.BlockSpec((16, H), lambda i: (0, 0)),
                  pl.BlockSpec((1, H), lambda i: (0, 0)),
                  pl.BlockSpec((2, H, H), lambda i: (0, 0, 0))],
        out_specs=[pl.BlockSpec((NBLK, H), lambda i: (i, 0)),
                   pl.BlockSpec((NBLK, H), lambda i: (i, 0)),
                   pl.BlockSpec((NBLK, H), lambda i: (i, 0))],
        out_shape=[jax.ShapeDtypeStruct((NPAD, H), F32),
                   jax.ShapeDtypeStruct((NPAD, H), F32),
                   jax.ShapeDtypeStruct((NPAD, H), F32)],
        interpret=interpret,
    )(x16, Wn16, bn, AB0)


def _basis_body(pg_ref, ea_ref, o_ref):
    ps = pg_ref[0]
    pd = pg_ref[1]
    d = pd - ps
    lane = lax.broadcasted_iota(jnp.int32, (EBLK, 16), 1)
    d = jnp.where(lane < 3, d, 0.0)
    lensq = jnp.sum(d * d, axis=1, keepdims=True)
    ln = jnp.sqrt(lensq + 1e-12)
    wdt = RMAX / (NBASIS - 1)
    centers = (lane.astype(F32) - 3.0) * wdt
    bas = jnp.exp(-0.5 * ((ln - centers) / wdt) ** 2)
    o_ref[...] = jnp.where(lane < 3, ea_ref[...],
                           jnp.where(lane < 11, bas, 0.0))


def _tc_basis(posg, ea16, interpret=False):
    return pl.pallas_call(
        _basis_body,
        grid=(E // EBLK,),
        in_specs=[pl.BlockSpec((2, EBLK, 16), lambda i: (0, i, 0)),
                  pl.BlockSpec((EBLK, 16), lambda i: (i, 0))],
        out_specs=pl.BlockSpec((EBLK, 16), lambda i: (i, 0)),
        out_shape=jax.ShapeDtypeStruct((E, 16), F32),
        interpret=interpret,
    )(posg, ea16)


def _efc_body(r_ref, d_ref, dv_ref, o_ref):
    o_ref[...] = (jnp.dot(r_ref[...], d_ref[...], preferred_element_type=F32)
                  + dv_ref[...])


def _tc_efc(raw, D16, dvec, interpret=False):
    return pl.pallas_call(
        _efc_body,
        grid=(E // EBLK,),
        in_specs=[pl.BlockSpec((EBLK, 16), lambda i: (i, 0)),
                  pl.BlockSpec((16, H), lambda i: (0, 0)),
                  pl.BlockSpec((1, H), lambda i: (0, 0))],
        out_specs=pl.BlockSpec((EBLK, H), lambda i: (i, 0)),
        out_shape=jax.ShapeDtypeStruct((E, H), F32),
        interpret=interpret,
    )(raw, D16, dvec)


def _update_body(h_ref, sp_ref, degp_ref, w_ref, v_ref, o_ref, oa_ref, ob_ref):
    h = h_ref[...]
    S = sp_ref[0] + sp_ref[1]
    deg = degp_ref[0][:, 0:1] + degp_ref[1][:, 0:1]
    W_m2 = w_ref[0]
    U1h = w_ref[1]
    U1a = w_ref[2]
    W_u2 = w_ref[3]
    b_m2 = v_ref[0:1]
    b_u1 = v_ref[1:2]
    b_u2 = v_ref[2:3]
    gam = v_ref[3:4]
    bet = v_ref[4:5]
    agg = jnp.dot(S, W_m2, preferred_element_type=F32) + deg * b_m2
    u = jnp.maximum(jnp.dot(h, U1h, preferred_element_type=F32)
                    + jnp.dot(agg, U1a, preferred_element_type=F32)
                    + b_u1, 0.0)
    out = jnp.dot(u, W_u2, preferred_element_type=F32) + b_u2 + h
    mu = jnp.mean(out, axis=1, keepdims=True)
    var = jnp.mean((out - mu) ** 2, axis=1, keepdims=True)
    hn = (out - mu) * lax.rsqrt(var + 1e-5) * gam + bet
    o_ref[...] = hn
    oa_ref[...] = jnp.dot(hn, w_ref[4], preferred_element_type=F32)
    ob_ref[...] = jnp.dot(hn, w_ref[5], preferred_element_type=F32)


def _tc_update(h, Sp, degp, Wm, vecs, interpret=False):
    return pl.pallas_call(
        _update_body,
        grid=(NPAD // NBLK,),
        in_specs=[pl.BlockSpec((NBLK, H), lambda i: (i, 0)),
                  pl.BlockSpec((2, NBLK, H), lambda i: (0, i, 0)),
                  pl.BlockSpec((2, NBLK, 16), lambda i: (0, i, 0)),
                  pl.BlockSpec((6, H, H), lambda i: (0, 0, 0)),
                  pl.BlockSpec((8, H), lambda i: (0, 0))],
        out_specs=[pl.BlockSpec((NBLK, H), lambda i: (i, 0)),
                   pl.BlockSpec((NBLK, H), lambda i: (i, 0)),
                   pl.BlockSpec((NBLK, H), lambda i: (i, 0))],
        out_shape=[jax.ShapeDtypeStruct((NPAD, H), F32),
                   jax.ShapeDtypeStruct((NPAD, H), F32),
                   jax.ShapeDtypeStruct((NPAD, H), F32)],
        interpret=interpret,
    )(h, Sp, degp, Wm, vecs)


def _head_body(h_ref, b3_ref, w_ref, v_ref, o_ref, acc_ref):
    i = pl.program_id(0)

    @pl.when(i == 0)
    def _init():
        acc_ref[...] = jnp.zeros_like(acc_ref)

    h = h_ref[...]
    b = b3_ref[0]
    m1 = (b == 0).astype(F32)
    m2 = (b == 1).astype(F32)
    acc_ref[0:1] += jnp.dot(m1, h, preferred_element_type=F32)
    acc_ref[1:2] += jnp.dot(m2, h, preferred_element_type=F32)
    acc_ref[2:3] = acc_ref[2:3] + jnp.sum(m1)
    acc_ref[3:4] = acc_ref[3:4] + jnp.sum(m2)

    @pl.when(i == NPAD // NBLK - 1)
    def _fin():
        c1 = jnp.maximum(acc_ref[2:3], 1.0)
        c2 = jnp.maximum(acc_ref[3:4], 1.0)
        s1 = acc_ref[0:1] / c1
        s2 = acc_ref[1:2] / c2
        hh = jnp.maximum(jnp.dot(s1, w_ref[0], preferred_element_type=F32)
                         + jnp.dot(s2, w_ref[1], preferred_element_type=F32)
                         + v_ref[0:1], 0.0)
        hh = jnp.maximum(jnp.dot(hh, w_ref[2], preferred_element_type=F32)
                         + v_ref[1:2], 0.0)
        z = jnp.dot(hh, w_ref[3], preferred_element_type=F32) + v_ref[2:3]
        o_ref[...] = jnp.maximum(z, 0.0) + jnp.log(1.0 + jnp.exp(-jnp.abs(z)))


def _tc_head(h, batch3, Whead, vhead, interpret=False):
    return pl.pallas_call(
        _head_body,
        grid=(NPAD // NBLK,),
        in_specs=[pl.BlockSpec((NBLK, H), lambda i: (i, 0)),
                  pl.BlockSpec((1, 1, NBLK), lambda i: (i, 0, 0)),
                  pl.BlockSpec((4, H, H), lambda i: (0, 0, 0)),
                  pl.BlockSpec((8, H), lambda i: (0, 0))],
        out_specs=pl.BlockSpec((1, H), lambda i: (0, 0)),
        out_shape=jax.ShapeDtypeStruct((1, H), F32),
        scratch_shapes=[pltpu.VMEM((8, H), F32)],
        interpret=interpret,
    )(h, batch3, Whead, vhead)


def _forward(x, edge_index, edge_attr, pos, batch, params, interpret=False):
    srcI = edge_index[0].astype(jnp.int32)
    dstI = edge_index[1].astype(jnp.int32)
    x16 = jnp.zeros((NPAD, 16), F32).at[:N, :6].set(x)
    pos16 = jnp.zeros((NPAD, 16), F32).at[:N, :3].set(pos)
    ea16 = jnp.zeros((E, 16), F32).at[:, :3].set(edge_attr)
    batchp = jnp.full((NPAD,), 2, jnp.int32).at[:N].set(batch.astype(jnp.int32))
    batch3 = batchp.reshape(NPAD // NBLK, 1, NBLK)
    zeros16 = jnp.zeros((NPAD, 16), F32)
    zeros128 = jnp.zeros((NPAD, H), F32)

    Wn16 = jnp.zeros((16, H), F32).at[:6].set(params['W_node'])
    bn = params['b_node'][None]
    We = params['W_edge']
    be = params['b_edge']
    z128 = jnp.zeros((H,), F32)
    zHH = jnp.zeros((H, H), F32)
    mp = params['mp']
    nl = len(mp)
    As = [lp['W_m1'][0:H] for lp in mp]
    Bs = [lp['W_m1'][H:2 * H] for lp in mp]
    Ds, dvs, Wms, vls = [], [], [], []
    for l, lp in enumerate(mp):
        C = lp['W_m1'][2 * H:]
        Ds.append(jnp.zeros((16, H), F32).at[:11].set(We @ C))
        dvs.append((be @ C + lp['b_m1'])[None])
        An = As[l + 1] if l + 1 < nl else zHH
        Bn = Bs[l + 1] if l + 1 < nl else zHH
        Wms.append(jnp.stack([lp['W_m2'], lp['W_u1'][0:H],
                              lp['W_u1'][H:], lp['W_u2'], An, Bn]))
        vls.append(jnp.stack([lp['b_m2'], lp['b_u1'], lp['b_u2'],
                              lp['gamma'], lp['beta'], z128, z128, z128]))
    D16s = jnp.stack(Ds)
    dvecs = jnp.stack(dvs)
    AB0 = jnp.stack([As[0], Bs[0]])
    eidx = jnp.stack([srcI.reshape(NWORK, NCHUNK, CB),
                      dstI.reshape(NWORK, NCHUNK, CB)], axis=2)
    Wh2 = jnp.zeros((H, H), F32).at[:, :H // 2].set(params['W_h2'])
    Wh3 = jnp.zeros((H, H), F32).at[:H // 2, 0:1].set(params['W_h3'])
    Whead = jnp.stack([params['W_h1'][0:H], params['W_h1'][H:], Wh2, Wh3])
    bh2p = jnp.zeros((H,), F32).at[:H // 2].set(params['b_h2'])
    vhead = jnp.stack([params['b_h1'], bh2p,
                       jnp.broadcast_to(params['b_h3'], (H,)),
                       z128, z128, z128, z128, z128])

    h, hA, hB = _tc_embed(x16, Wn16, bn, AB0, interpret)
    posg, degp = _sc_prologue(pos16, eidx, zeros16, interpret)
    raw = _tc_basis(posg, ea16, interpret)
    efc = _tc_efc(raw, D16s[0], dvecs[0], interpret)
    for l in range(nl):
        Sp = _sc_edge(hA, hB, efc, eidx, zeros128, 0, interpret)
        if l + 1 < nl:
            efc = _tc_efc(raw, D16s[l + 1], dvecs[l + 1], interpret)
        h, hA, hB = _tc_update(h, Sp, degp, Wms[l], vls[l], interpret)
    out = _tc_head(h, batch3, Whead, vhead, interpret)
    return out[0, 0:1]


def kernel(x, edge_index, edge_attr, pos, batch, params):
    return _forward(x, edge_index, edge_attr, pos, batch, params)
